# baseline (speedup 1.0000x reference)
import functools
import math

import jax
import jax.numpy as jnp
from jax.experimental import pallas as pl
from jax.experimental.pallas import tpu as pltpu
from jax.experimental.pallas import tpu_sc as plsc

VOCAB = 32000
D = 512
NL = 4
NH = 8
DH = 64
T = 1024
B = 2
R = B * T
RB = 256
NR = R // RB
FF = 2048
TOPK = 4

VBLK_E = 1280
NV_E = VOCAB // VBLK_E
VBLK_L = 3200
NV_L = VOCAB // VBLK_L
SBLK = 2048

_NEG = float("-inf")
_BIGI = 2**30

_NC = 2
_NS = 16
_NW = _NC * _NS
_GCH = 64


def _lnf(x, s, b):
    m = jnp.mean(x, axis=1, keepdims=True)
    v = jnp.mean((x - m) ** 2, axis=1, keepdims=True)
    return (x - m) / jnp.sqrt(v + 1e-5) * s + b


def _sc_gather(idx, table):
    n = idx.shape[0]
    epw = n // _NW
    nch = epw // _GCH
    mesh = plsc.VectorSubcoreMesh(core_axis_name="c", subcore_axis_name="s",
                                  num_cores=_NC, num_subcores=_NS)

    @functools.partial(
        pl.kernel,
        mesh=mesh,
        out_type=jax.ShapeDtypeStruct((n, D), jnp.float32),
        scratch_types=[
            pltpu.VMEM((_GCH,), jnp.int32),
            pltpu.VMEM((_GCH, D), jnp.float32),
            pltpu.SemaphoreType.DMA,
        ],
    )
    def k(table_hbm, idx_hbm, out_hbm, idx_v, rows_v, sem):
        wid = jax.lax.axis_index("s") * _NC + jax.lax.axis_index("c")
        base = wid * epw
        for c in range(nch):
            off = base + c * _GCH
            pltpu.sync_copy(idx_hbm.at[pl.ds(off, _GCH)], idx_v)
            pltpu.async_copy(table_hbm.at[idx_v], rows_v, sem).wait()
            pltpu.sync_copy(rows_v, out_hbm.at[pl.ds(off, _GCH)])

    return k(table, idx)


def _addpos_body(g_ref, p_ref, out_ref):
    out_ref[...] = g_ref[...] + p_ref[...]


def _addpos(g, pos):
    npb = T // RB
    return pl.pallas_call(
        _addpos_body,
        grid=(NR,),
        in_specs=[
            pl.BlockSpec((RB, D), lambda r: (r, 0)),
            pl.BlockSpec((RB, D), lambda r: (r % npb, 0)),
        ],
        out_specs=pl.BlockSpec((RB, D), lambda r: (r, 0)),
        out_shape=jax.ShapeDtypeStruct((R, D), jnp.float32),
    )(g, pos)


def _qkv_body(x_ref, s_ref, b_ref, w_ref, bias_ref, out_ref):
    h = _lnf(x_ref[...], s_ref[...], b_ref[...]).astype(jnp.bfloat16)
    out_ref[...] = jnp.dot(h, w_ref[...], preferred_element_type=jnp.float32) + bias_ref[...]


def _qkv(x, s, b, w, bias):
    return pl.pallas_call(
        _qkv_body,
        grid=(NR,),
        in_specs=[
            pl.BlockSpec((RB, D), lambda r: (r, 0)),
            pl.BlockSpec((1, D), lambda r: (0, 0)),
            pl.BlockSpec((1, D), lambda r: (0, 0)),
            pl.BlockSpec((D, 3 * D), lambda r: (0, 0)),
            pl.BlockSpec((1, 3 * D), lambda r: (0, 0)),
        ],
        out_specs=pl.BlockSpec((RB, 3 * D), lambda r: (r, 0)),
        out_shape=jax.ShapeDtypeStruct((R, 3 * D), jnp.float32),
    )(x, s, b, w, bias)


def _attn_body(qkv_ref, o_ref):
    ri = jax.lax.broadcasted_iota(jnp.int32, (T, T), 0)
    ci = jax.lax.broadcasted_iota(jnp.int32, (T, T), 1)
    causal = ri >= ci
    for h in range(NH):
        q = qkv_ref[:, h * DH:(h + 1) * DH].astype(jnp.bfloat16)
        k = qkv_ref[:, D + h * DH:D + (h + 1) * DH].astype(jnp.bfloat16)
        v = qkv_ref[:, 2 * D + h * DH:2 * D + (h + 1) * DH].astype(jnp.bfloat16)
        s = jax.lax.dot_general(q, k, (((1,), (1,)), ((), ())),
                                preferred_element_type=jnp.float32)
        s = s * (1.0 / math.sqrt(DH))
        s = jnp.where(causal, s, jnp.float32(-1e9))
        m = jnp.max(s, axis=1, keepdims=True)
        e = jnp.exp(s - m)
        p = (e / jnp.sum(e, axis=1, keepdims=True)).astype(jnp.bfloat16)
        o_ref[:, h * DH:(h + 1) * DH] = jnp.dot(
            p, v, preferred_element_type=jnp.float32)


def _attn(qkv):
    return pl.pallas_call(
        _attn_body,
        grid=(B,),
        in_specs=[pl.BlockSpec((T, 3 * D), lambda b: (b, 0))],
        out_specs=pl.BlockSpec((T, D), lambda b: (b, 0)),
        out_shape=jax.ShapeDtypeStruct((R, D), jnp.float32),
    )(qkv)


def _post_body(x_ref, o_ref, Wo_ref, bo_ref, s2_ref, b2_ref, W1_ref, b1_ref,
               W2_ref, b2m_ref, out_ref):
    x = x_ref[...] + jnp.dot(o_ref[...].astype(jnp.bfloat16), Wo_ref[...],
                             preferred_element_type=jnp.float32) + bo_ref[...]
    h2 = _lnf(x, s2_ref[...], b2_ref[...]).astype(jnp.bfloat16)
    ff = jax.nn.gelu(jnp.dot(h2, W1_ref[...],
                             preferred_element_type=jnp.float32) + b1_ref[...])
    out_ref[...] = x + jnp.dot(ff.astype(jnp.bfloat16), W2_ref[...],
                               preferred_element_type=jnp.float32) + b2m_ref[...]


def _post(x, o, Wo_l, bo_l, s2, b2, W1_l, b1_l, W2_l, b2m):
    return pl.pallas_call(
        _post_body,
        grid=(NR,),
        in_specs=[
            pl.BlockSpec((RB, D), lambda r: (r, 0)),
            pl.BlockSpec((RB, D), lambda r: (r, 0)),
            pl.BlockSpec((D, D), lambda r: (0, 0)),
            pl.BlockSpec((1, D), lambda r: (0, 0)),
            pl.BlockSpec((1, D), lambda r: (0, 0)),
            pl.BlockSpec((1, D), lambda r: (0, 0)),
            pl.BlockSpec((D, FF), lambda r: (0, 0)),
            pl.BlockSpec((1, FF), lambda r: (0, 0)),
            pl.BlockSpec((FF, D), lambda r: (0, 0)),
            pl.BlockSpec((1, D), lambda r: (0, 0)),
        ],
        out_specs=pl.BlockSpec((RB, D), lambda r: (r, 0)),
        out_shape=jax.ShapeDtypeStruct((R, D), jnp.float32),
    )(x, o, Wo_l, bo_l, s2, b2, W1_l, b1_l, W2_l, b2m)


def _rowmm_body(x_ref, w_ref, b_ref, out_ref):
    out_ref[...] = jnp.dot(x_ref[...], w_ref[...],
                           preferred_element_type=jnp.float32) + b_ref[...]


def _rowmm(x, w, b):
    return pl.pallas_call(
        _rowmm_body,
        grid=(NR,),
        in_specs=[
            pl.BlockSpec((RB, D), lambda r: (r, 0)),
            pl.BlockSpec((D, D), lambda r: (0, 0)),
            pl.BlockSpec((1, D), lambda r: (0, 0)),
        ],
        out_specs=pl.BlockSpec((RB, D), lambda r: (r, 0)),
        out_shape=jax.ShapeDtypeStruct((R, D), jnp.float32),
    )(x, w, b)


def _topk_body(qm_ref, K_ref, Sb_ref, tv_ref, ti_ref, sv, si, *, ns):
    s_idx = pl.program_id(0)
    r_idx = pl.program_id(1)
    sc = jax.lax.dot_general(qm_ref[...], K_ref[...], (((1,), (1,)), ((), ())),
                             preferred_element_type=jnp.float32)
    sc = sc * (1.0 / math.sqrt(D)) + Sb_ref[...]
    iota = jax.lax.broadcasted_iota(jnp.int32, (RB, SBLK), 1) + s_idx * SBLK
    bvs, bis = [], []
    for _ in range(TOPK):
        m = jnp.max(sc, axis=1, keepdims=True)
        mi = jnp.min(jnp.where(sc == m, iota, _BIGI), axis=1, keepdims=True)
        bvs.append(m)
        bis.append(mi)
        sc = jnp.where(iota == mi, _NEG, sc)
    bv = jnp.concatenate(bvs, axis=1)
    bi = jnp.concatenate(bis, axis=1)

    rsl = pl.ds(r_idx * RB, RB)

    @pl.when(s_idx == 0)
    def _():
        sv[rsl, :] = jnp.full((RB, TOPK), _NEG, jnp.float32)
        si[rsl, :] = jnp.zeros((RB, TOPK), jnp.int32)

    av = jnp.concatenate([sv[rsl, :], bv], axis=1)
    ai = jnp.concatenate([si[rsl, :], bi], axis=1)
    nvs, nis = [], []
    for _ in range(TOPK):
        m = jnp.max(av, axis=1, keepdims=True)
        mi = jnp.min(jnp.where(av == m, ai, _BIGI), axis=1, keepdims=True)
        nvs.append(m)
        nis.append(mi)
        av = jnp.where(ai == mi, _NEG, av)
    sv[rsl, :] = jnp.concatenate(nvs, axis=1)
    si[rsl, :] = jnp.concatenate(nis, axis=1)

    @pl.when(s_idx == ns - 1)
    def _():
        tv_ref[...] = sv[rsl, :]
        ti_ref[...] = si[rsl, :]


def _topk(qm, K, Sb):
    S = K.shape[0]
    ns = S // SBLK
    return pl.pallas_call(
        functools.partial(_topk_body, ns=ns),
        grid=(ns, NR),
        in_specs=[
            pl.BlockSpec((RB, D), lambda s, r: (r, 0)),
            pl.BlockSpec((SBLK, D), lambda s, r: (s, 0)),
            pl.BlockSpec((1, SBLK), lambda s, r: (0, s)),
        ],
        out_specs=[
            pl.BlockSpec((RB, TOPK), lambda s, r: (r, 0)),
            pl.BlockSpec((RB, TOPK), lambda s, r: (r, 0)),
        ],
        out_shape=[
            jax.ShapeDtypeStruct((R, TOPK), jnp.float32),
            jax.ShapeDtypeStruct((R, TOPK), jnp.int32),
        ],
        scratch_shapes=[
            pltpu.VMEM((R, TOPK), jnp.float32),
            pltpu.VMEM((R, TOPK), jnp.int32),
        ],
    )(qm, K, Sb)


def _final_body(x_ref, g0_ref, g1_ref, g2_ref, tv0_ref, tv1_ref, tv2_ref,
                Wr_ref, br_ref, s_ref, b_ref, out_ref):
    rd = jnp.zeros((RB, D), jnp.float32)
    for g_ref, tv_ref in ((g0_ref, tv0_ref), (g1_ref, tv1_ref),
                          (g2_ref, tv2_ref)):
        tv = tv_ref[...]
        m = jnp.max(tv, axis=1, keepdims=True)
        e = jnp.exp(tv - m)
        a = e / jnp.sum(e, axis=1, keepdims=True) * (1.0 / 3.0)
        for kk in range(TOPK):
            rd = rd + a[:, kk:kk + 1] * g_ref[kk]
    x = x_ref[...] + jnp.dot(rd, Wr_ref[...],
                             preferred_element_type=jnp.float32) + br_ref[...]
    out_ref[...] = _lnf(x, s_ref[...], b_ref[...])


def _final(x, g0, g1, g2, tv0, tv1, tv2, Wr, br, s, b):
    gspec = pl.BlockSpec((TOPK, RB, D), lambda r: (0, r, 0))
    tspec = pl.BlockSpec((RB, TOPK), lambda r: (r, 0))
    return pl.pallas_call(
        _final_body,
        grid=(NR,),
        in_specs=[
            pl.BlockSpec((RB, D), lambda r: (r, 0)),
            gspec, gspec, gspec, tspec, tspec, tspec,
            pl.BlockSpec((D, D), lambda r: (0, 0)),
            pl.BlockSpec((1, D), lambda r: (0, 0)),
            pl.BlockSpec((1, D), lambda r: (0, 0)),
            pl.BlockSpec((1, D), lambda r: (0, 0)),
        ],
        out_specs=pl.BlockSpec((RB, D), lambda r: (r, 0)),
        out_shape=jax.ShapeDtypeStruct((R, D), jnp.float32),
    )(x, g0, g1, g2, tv0, tv1, tv2, Wr, br, s, b)


def _logits_body(x_ref, emb_ref, out_ref):
    out_ref[...] = jax.lax.dot_general(
        x_ref[...].astype(jnp.bfloat16), emb_ref[...], (((1,), (1,)), ((), ())),
        preferred_element_type=jnp.float32)


def _logits(x, emb):
    return pl.pallas_call(
        _logits_body,
        grid=(NV_L, NR),
        in_specs=[
            pl.BlockSpec((RB, D), lambda v, r: (r, 0)),
            pl.BlockSpec((VBLK_L, D), lambda v, r: (v, 0)),
        ],
        out_specs=pl.BlockSpec((RB, VBLK_L), lambda v, r: (r, v)),
        out_shape=jax.ShapeDtypeStruct((R, VOCAB), jnp.float32),
    )(x, emb)


def kernel(input_ids, tok_embed, pos_embed, ln1_s, ln1_b, Wqkv, bqkv, Wo, bo,
           ln2_s, ln2_b, W1, b1, W2, b2, Wq_mem, bq_mem, Wr_mem, br_mem,
           out_s, out_b, K0, V0, S0, K1, V1, S1, K2, V2, S2):
    ids = input_ids.reshape(R).astype(jnp.int32)
    Wqkv = Wqkv.astype(jnp.bfloat16)
    Wo = Wo.astype(jnp.bfloat16)
    W1 = W1.astype(jnp.bfloat16)
    W2 = W2.astype(jnp.bfloat16)
    emb_bf = tok_embed.astype(jnp.bfloat16)
    x = _addpos(_sc_gather(ids, tok_embed), pos_embed[:T])
    for l in range(NL):
        qkv = _qkv(x, ln1_s[l].reshape(1, D), ln1_b[l].reshape(1, D),
                   Wqkv[l], bqkv[l].reshape(1, 3 * D))
        o = _attn(qkv)
        x = _post(x, o, Wo[l], bo[l].reshape(1, D), ln2_s[l].reshape(1, D),
                  ln2_b[l].reshape(1, D), W1[l], b1[l].reshape(1, FF),
                  W2[l], b2[l].reshape(1, D))
    qm = _rowmm(x, Wq_mem, bq_mem.reshape(1, D))
    tvs, gs = [], []
    for Ki, Vi, Si in ((K0, V0, S0), (K1, V1, S1), (K2, V2, S2)):
        tv, ti = _topk(qm, Ki, Si.reshape(1, -1))
        tvs.append(tv)
        idxk = ti.T.reshape(TOPK * R)
        gs.append(_sc_gather(idxk, Vi).reshape(TOPK, R, D))
    xf = _final(x, gs[0], gs[1], gs[2], tvs[0], tvs[1], tvs[2], Wr_mem,
                br_mem.reshape(1, D), out_s.reshape(1, D), out_b.reshape(1, D))
    logits = _logits(xf, emb_bf)
    return logits.reshape(B, T, VOCAB)


# speedup vs baseline: 1.1327x; 1.1327x over previous
import functools
import math

import jax
import jax.numpy as jnp
from jax.experimental import pallas as pl
from jax.experimental.pallas import tpu as pltpu
from jax.experimental.pallas import tpu_sc as plsc

VOCAB = 32000
D = 512
NL = 4
NH = 8
DH = 64
T = 1024
B = 2
R = B * T
RB = 256
NR = R // RB
FF = 2048
TOPK = 4

VBLK_E = 1280
NV_E = VOCAB // VBLK_E
VBLK_L = 3200
NV_L = VOCAB // VBLK_L
SBLK = 2048

_NEG = float("-inf")
_BIGI = 2**30

_NC = 2
_NS = 16
_NW = _NC * _NS
_GCH = 64


def _lnf(x, s, b):
    m = jnp.mean(x, axis=1, keepdims=True)
    v = jnp.mean((x - m) ** 2, axis=1, keepdims=True)
    return (x - m) / jnp.sqrt(v + 1e-5) * s + b


def _sc_gather(idx, table):
    n = idx.shape[0]
    epw = n // _NW
    nch = epw // _GCH
    mesh = plsc.VectorSubcoreMesh(core_axis_name="c", subcore_axis_name="s",
                                  num_cores=_NC, num_subcores=_NS)

    @functools.partial(
        pl.kernel,
        mesh=mesh,
        out_type=jax.ShapeDtypeStruct((n, D), jnp.float32),
        scratch_types=[
            pltpu.VMEM((_GCH,), jnp.int32),
            pltpu.VMEM((_GCH, D), jnp.float32),
            pltpu.SemaphoreType.DMA,
        ],
    )
    def k(table_hbm, idx_hbm, out_hbm, idx_v, rows_v, sem):
        wid = jax.lax.axis_index("s") * _NC + jax.lax.axis_index("c")
        base = wid * epw
        for c in range(nch):
            off = base + c * _GCH
            pltpu.sync_copy(idx_hbm.at[pl.ds(off, _GCH)], idx_v)
            pltpu.async_copy(table_hbm.at[idx_v], rows_v, sem).wait()
            pltpu.sync_copy(rows_v, out_hbm.at[pl.ds(off, _GCH)])

    return k(table, idx)


def _layers_body(xg_ref, pos_ref, ln1s_ref, ln1b_ref, Wqkv_ref, bqkv_ref,
                 Wo_ref, bo_ref, ln2s_ref, ln2b_ref, W1_ref, b1_ref,
                 W2_ref, b2m_ref, Wq_ref, bq_ref, xout_ref, qm_ref, xs):
    l = pl.program_id(0)
    b = pl.program_id(1)
    rsl = pl.ds(b * T, T)

    @pl.when(l == 0)
    def _():
        xs[rsl, :] = xg_ref[...] + pos_ref[...]

    x = xs[rsl, :]
    h = _lnf(x, ln1s_ref[0], ln1b_ref[0]).astype(jnp.bfloat16)
    qkv = jnp.dot(h, Wqkv_ref[0], preferred_element_type=jnp.float32)
    qkv = qkv + bqkv_ref[0]

    ri = jax.lax.broadcasted_iota(jnp.int32, (T, T), 0)
    ci = jax.lax.broadcasted_iota(jnp.int32, (T, T), 1)
    causal = ri >= ci
    ohs = []
    for hh in range(NH):
        q = qkv[:, hh * DH:(hh + 1) * DH].astype(jnp.bfloat16)
        k = qkv[:, D + hh * DH:D + (hh + 1) * DH].astype(jnp.bfloat16)
        v = qkv[:, 2 * D + hh * DH:2 * D + (hh + 1) * DH].astype(jnp.bfloat16)
        s = jax.lax.dot_general(q, k, (((1,), (1,)), ((), ())),
                                preferred_element_type=jnp.float32)
        s = s * (1.0 / math.sqrt(DH))
        s = jnp.where(causal, s, jnp.float32(-1e9))
        m = jnp.max(s, axis=1, keepdims=True)
        e = jnp.exp(s - m)
        p = (e / jnp.sum(e, axis=1, keepdims=True)).astype(jnp.bfloat16)
        ohs.append(jnp.dot(p, v, preferred_element_type=jnp.float32))
    o = jnp.concatenate(ohs, axis=1)

    x = x + jnp.dot(o.astype(jnp.bfloat16), Wo_ref[0],
                    preferred_element_type=jnp.float32) + bo_ref[0]
    h2 = _lnf(x, ln2s_ref[0], ln2b_ref[0]).astype(jnp.bfloat16)
    ff = jax.nn.gelu(jnp.dot(h2, W1_ref[0],
                             preferred_element_type=jnp.float32) + b1_ref[0])
    x = x + jnp.dot(ff.astype(jnp.bfloat16), W2_ref[0],
                    preferred_element_type=jnp.float32) + b2m_ref[0]
    xs[rsl, :] = x

    @pl.when(l == NL - 1)
    def _():
        xout_ref[...] = x
        qm_ref[...] = jnp.dot(x, Wq_ref[...],
                              preferred_element_type=jnp.float32) + bq_ref[...]


def _layers(xg, pos, ln1_s, ln1_b, Wqkv, bqkv, Wo, bo, ln2_s, ln2_b,
            W1, b1, W2, b2, Wq, bq):
    lvec = lambda n: pl.BlockSpec((1, 1, n), lambda l, b: (l, 0, 0))
    lmat = lambda m, n: pl.BlockSpec((1, m, n), lambda l, b: (l, 0, 0))
    return pl.pallas_call(
        _layers_body,
        grid=(NL, B),
        in_specs=[
            pl.BlockSpec((T, D), lambda l, b: (b, 0)),
            pl.BlockSpec((T, D), lambda l, b: (0, 0)),
            lvec(D), lvec(D),
            lmat(D, 3 * D), lvec(3 * D),
            lmat(D, D), lvec(D),
            lvec(D), lvec(D),
            lmat(D, FF), lvec(FF),
            lmat(FF, D), lvec(D),
            pl.BlockSpec((D, D), lambda l, b: (0, 0)),
            pl.BlockSpec((1, D), lambda l, b: (0, 0)),
        ],
        out_specs=[
            pl.BlockSpec((T, D), lambda l, b: (b, 0)),
            pl.BlockSpec((T, D), lambda l, b: (b, 0)),
        ],
        out_shape=[
            jax.ShapeDtypeStruct((R, D), jnp.float32),
            jax.ShapeDtypeStruct((R, D), jnp.float32),
        ],
        scratch_shapes=[pltpu.VMEM((R, D), jnp.float32)],
    )(xg, pos, ln1_s, ln1_b, Wqkv, bqkv, Wo, bo, ln2_s, ln2_b,
      W1, b1, W2, b2, Wq, bq)


def _topk_body(qm_ref, K_ref, Sb_ref, tv_ref, ti_ref, sv, si, *, ns):
    s_idx = pl.program_id(0)
    r_idx = pl.program_id(1)
    sc = jax.lax.dot_general(qm_ref[...], K_ref[...], (((1,), (1,)), ((), ())),
                             preferred_element_type=jnp.float32)
    sc = sc * (1.0 / math.sqrt(D)) + Sb_ref[...]
    iota = jax.lax.broadcasted_iota(jnp.int32, (RB, SBLK), 1) + s_idx * SBLK
    bvs, bis = [], []
    for _ in range(TOPK):
        m = jnp.max(sc, axis=1, keepdims=True)
        mi = jnp.min(jnp.where(sc == m, iota, _BIGI), axis=1, keepdims=True)
        bvs.append(m)
        bis.append(mi)
        sc = jnp.where(iota == mi, _NEG, sc)
    bv = jnp.concatenate(bvs, axis=1)
    bi = jnp.concatenate(bis, axis=1)

    rsl = pl.ds(r_idx * RB, RB)

    @pl.when(s_idx == 0)
    def _():
        sv[rsl, :] = jnp.full((RB, TOPK), _NEG, jnp.float32)
        si[rsl, :] = jnp.zeros((RB, TOPK), jnp.int32)

    av = jnp.concatenate([sv[rsl, :], bv], axis=1)
    ai = jnp.concatenate([si[rsl, :], bi], axis=1)
    nvs, nis = [], []
    for _ in range(TOPK):
        m = jnp.max(av, axis=1, keepdims=True)
        mi = jnp.min(jnp.where(av == m, ai, _BIGI), axis=1, keepdims=True)
        nvs.append(m)
        nis.append(mi)
        av = jnp.where(ai == mi, _NEG, av)
    sv[rsl, :] = jnp.concatenate(nvs, axis=1)
    si[rsl, :] = jnp.concatenate(nis, axis=1)

    @pl.when(s_idx == ns - 1)
    def _():
        tv_ref[...] = sv[rsl, :]
        ti_ref[...] = si[rsl, :]


def _topk(qm, K, Sb):
    S = K.shape[0]
    ns = S // SBLK
    return pl.pallas_call(
        functools.partial(_topk_body, ns=ns),
        grid=(ns, NR),
        in_specs=[
            pl.BlockSpec((RB, D), lambda s, r: (r, 0)),
            pl.BlockSpec((SBLK, D), lambda s, r: (s, 0)),
            pl.BlockSpec((1, SBLK), lambda s, r: (0, s)),
        ],
        out_specs=[
            pl.BlockSpec((RB, TOPK), lambda s, r: (r, 0)),
            pl.BlockSpec((RB, TOPK), lambda s, r: (r, 0)),
        ],
        out_shape=[
            jax.ShapeDtypeStruct((R, TOPK), jnp.float32),
            jax.ShapeDtypeStruct((R, TOPK), jnp.int32),
        ],
        scratch_shapes=[
            pltpu.VMEM((R, TOPK), jnp.float32),
            pltpu.VMEM((R, TOPK), jnp.int32),
        ],
    )(qm, K, Sb)


def _final_body(x_ref, g0_ref, g1_ref, g2_ref, tv0_ref, tv1_ref, tv2_ref,
                Wr_ref, br_ref, s_ref, b_ref, out_ref):
    rd = jnp.zeros((RB, D), jnp.float32)
    for g_ref, tv_ref in ((g0_ref, tv0_ref), (g1_ref, tv1_ref),
                          (g2_ref, tv2_ref)):
        tv = tv_ref[...]
        m = jnp.max(tv, axis=1, keepdims=True)
        e = jnp.exp(tv - m)
        a = e / jnp.sum(e, axis=1, keepdims=True) * (1.0 / 3.0)
        for kk in range(TOPK):
            rd = rd + a[:, kk:kk + 1] * g_ref[kk]
    x = x_ref[...] + jnp.dot(rd, Wr_ref[...],
                             preferred_element_type=jnp.float32) + br_ref[...]
    out_ref[...] = _lnf(x, s_ref[...], b_ref[...])


def _final(x, g0, g1, g2, tv0, tv1, tv2, Wr, br, s, b):
    gspec = pl.BlockSpec((TOPK, RB, D), lambda r: (0, r, 0))
    tspec = pl.BlockSpec((RB, TOPK), lambda r: (r, 0))
    return pl.pallas_call(
        _final_body,
        grid=(NR,),
        in_specs=[
            pl.BlockSpec((RB, D), lambda r: (r, 0)),
            gspec, gspec, gspec, tspec, tspec, tspec,
            pl.BlockSpec((D, D), lambda r: (0, 0)),
            pl.BlockSpec((1, D), lambda r: (0, 0)),
            pl.BlockSpec((1, D), lambda r: (0, 0)),
            pl.BlockSpec((1, D), lambda r: (0, 0)),
        ],
        out_specs=pl.BlockSpec((RB, D), lambda r: (r, 0)),
        out_shape=jax.ShapeDtypeStruct((R, D), jnp.float32),
    )(x, g0, g1, g2, tv0, tv1, tv2, Wr, br, s, b)


def _logits_body(x_ref, emb_ref, out_ref):
    out_ref[...] = jax.lax.dot_general(
        x_ref[...].astype(jnp.bfloat16), emb_ref[...], (((1,), (1,)), ((), ())),
        preferred_element_type=jnp.float32)


def _logits(x, emb):
    return pl.pallas_call(
        _logits_body,
        grid=(NV_L, NR),
        in_specs=[
            pl.BlockSpec((RB, D), lambda v, r: (r, 0)),
            pl.BlockSpec((VBLK_L, D), lambda v, r: (v, 0)),
        ],
        out_specs=pl.BlockSpec((RB, VBLK_L), lambda v, r: (r, v)),
        out_shape=jax.ShapeDtypeStruct((R, VOCAB), jnp.float32),
    )(x, emb)


def kernel(input_ids, tok_embed, pos_embed, ln1_s, ln1_b, Wqkv, bqkv, Wo, bo,
           ln2_s, ln2_b, W1, b1, W2, b2, Wq_mem, bq_mem, Wr_mem, br_mem,
           out_s, out_b, K0, V0, S0, K1, V1, S1, K2, V2, S2):
    ids = input_ids.reshape(R).astype(jnp.int32)
    Wqkv = Wqkv.astype(jnp.bfloat16)
    Wo = Wo.astype(jnp.bfloat16)
    W1 = W1.astype(jnp.bfloat16)
    W2 = W2.astype(jnp.bfloat16)
    emb_bf = tok_embed.astype(jnp.bfloat16)
    xg = _sc_gather(ids, tok_embed)
    x, qm = _layers(xg, pos_embed[:T],
                    ln1_s.reshape(NL, 1, D), ln1_b.reshape(NL, 1, D),
                    Wqkv, bqkv.reshape(NL, 1, 3 * D),
                    Wo, bo.reshape(NL, 1, D),
                    ln2_s.reshape(NL, 1, D), ln2_b.reshape(NL, 1, D),
                    W1, b1.reshape(NL, 1, FF),
                    W2, b2.reshape(NL, 1, D),
                    Wq_mem, bq_mem.reshape(1, D))
    tvs, gs = [], []
    for Ki, Vi, Si in ((K0, V0, S0), (K1, V1, S1), (K2, V2, S2)):
        tv, ti = _topk(qm, Ki, Si.reshape(1, -1))
        tvs.append(tv)
        idxk = ti.T.reshape(TOPK * R)
        gs.append(_sc_gather(idxk, Vi).reshape(TOPK, R, D))
    xf = _final(x, gs[0], gs[1], gs[2], tvs[0], tvs[1], tvs[2], Wr_mem,
                br_mem.reshape(1, D), out_s.reshape(1, D), out_b.reshape(1, D))
    logits = _logits(xf, emb_bf)
    return logits.reshape(B, T, VOCAB)
